# Initial kernel scaffold; baseline (speedup 1.0000x reference)
import jax
import jax.numpy as jnp
from jax.experimental import pallas as pl


def kernel(x, edge_index, src, dst, W1, a_src1, a_dst1, b1, W2, a_src2, a_dst2, b2, Wl, bl):
    raise NotImplementedError("write your pallas kernel here")


# speedup vs baseline: 1191.2469x; 1191.2469x over previous
import functools

import jax
import jax.numpy as jnp
from jax import lax
from jax.experimental import pallas as pl
from jax.experimental.pallas import tpu as pltpu
from jax.experimental.pallas import tpu_sc as plsc

N = 100000
E = 3200000
NP = 100352
RP = 784
NW = 32
EPT = E // NW
CH = 4000
NG = CH // 16
ACC = 6 * NP
SLC = ACC // 16
ZB = SLC // 3


def _tc_prep(xpad, W1, as1, ad1):
    def body(x_ref, w_ref, as_ref, ad_ref, out_ref):
        X = jnp.max(jnp.abs(x_ref[...]))
        rows = []
        cs = []
        cd = []
        for h in range(2):
            csh = sum(w_ref[0, 4 * h + c] * as_ref[h, c] for c in range(4))
            cdh = sum(w_ref[0, 4 * h + c] * ad_ref[h, c] for c in range(4))
            cs.append(csh)
            cd.append(cdh)
        vals = [cs[0], cs[1], cd[0], cd[1],
                X * jnp.abs(cs[0]), X * jnp.abs(cs[1]), X, 0.0]
        rows = [jnp.full((16,), v, jnp.float32) for v in vals]
        out_ref[...] = jnp.stack(rows)

    return pl.pallas_call(
        body,
        out_shape=jax.ShapeDtypeStruct((8, 16), jnp.float32),
    )(xpad, W1, as1, ad1)


def _sc_edges(es, ed, xf, consts, sd2):
    mesh = plsc.VectorSubcoreMesh(core_axis_name="c", subcore_axis_name="s")

    @functools.partial(
        pl.kernel,
        mesh=mesh,
        out_type=jax.ShapeDtypeStruct((2 * ACC,), jnp.float32),
        scratch_types=[
            pltpu.VMEM_SHARED((ACC,), jnp.float32),
            pltpu.VMEM((CH,), jnp.int32),
            pltpu.VMEM((CH,), jnp.int32),
            pltpu.VMEM((CH,), jnp.float32),
            pltpu.VMEM((CH,), jnp.float32),
            pltpu.VMEM((6 * CH,), jnp.int32),
            pltpu.VMEM((6 * CH,), jnp.float32),
            pltpu.VMEM((ZB,), jnp.float32),
            pltpu.VMEM((8, 16), jnp.float32),
            pltpu.VMEM((2, 16), jnp.int32),
            pltpu.SemaphoreType.DMA,
        ],
    )
    def k(es_h, ed_h, xf_h, consts_h, sd2_h, out_h,
          acc, es_v, ed_v, xs_v, xd_v, idx_v, val_v, zb_v, cv, sdv, sem):
        cid = lax.axis_index("c")
        sid = lax.axis_index("s")
        wid = cid * 16 + sid

        def zb_body(i, _):
            zb_v[pl.ds(i * 16, 16)] = jnp.zeros((16,), jnp.float32)
            return 0
        lax.fori_loop(0, ZB // 16, zb_body, 0)
        for kk in range(3):
            pltpu.sync_copy(zb_v, acc.at[pl.ds(sid * SLC + kk * ZB, ZB)])

        pltpu.sync_copy(consts_h, cv)
        pltpu.sync_copy(sd2_h, sdv)
        plsc.subcore_barrier()

        a0 = cv[0, :]
        a1 = cv[1, :]
        b0 = cv[2, :]
        b1 = cv[3, :]
        c0 = cv[4, :]
        c1 = cv[5, :]
        srcv = sdv[0, :]
        dstv = sdv[1, :]

        def chunk_body(ci, _):
            base = wid * EPT + ci * CH
            pltpu.sync_copy(es_h.at[pl.ds(base, CH)], es_v)
            pltpu.sync_copy(ed_h.at[pl.ds(base, CH)], ed_v)
            pltpu.async_copy(xf_h.at[es_v], xs_v, sem).wait()
            pltpu.async_copy(xf_h.at[ed_v], xd_v, sem).wait()

            def group_body(g, _):
                sl = pl.ds(g * 16, 16)
                s = es_v[sl]
                d = ed_v[sl]
                xs = xs_v[sl]
                xd = xd_v[sl]
                one = jnp.full((16,), 1.0, jnp.float32)
                zero = jnp.zeros((16,), jnp.float32)
                for h, (ah, bh, chh) in enumerate(((a0, b0, c0), (a1, b1, c1))):
                    xdb = xd * bh
                    arg = xs * ah + xdb
                    e = jnp.maximum(arg, 0.2 * arg)
                    ma = chh + xdb
                    m = jnp.maximum(ma, 0.2 * ma)
                    p = jnp.exp(e - m)
                    idx_v[pl.ds(h * CH + g * 16, 16)] = d + h * NP
                    val_v[pl.ds(h * CH + g * 16, 16)] = p
                    idx_v[pl.ds((2 + h) * CH + g * 16, 16)] = d + (2 + h) * NP
                    val_v[pl.ds((2 + h) * CH + g * 16, 16)] = p * xs
                idx_v[pl.ds(4 * CH + g * 16, 16)] = s + 4 * NP
                val_v[pl.ds(4 * CH + g * 16, 16)] = jnp.where(d == srcv, one, zero)
                idx_v[pl.ds(5 * CH + g * 16, 16)] = s + 5 * NP
                val_v[pl.ds(5 * CH + g * 16, 16)] = jnp.where(d == dstv, one, zero)
                return 0

            lax.fori_loop(0, NG, group_body, 0)
            pltpu.sync_copy(val_v, acc.at[idx_v], add=True)
            return 0

        lax.fori_loop(0, EPT // CH, chunk_body, 0)
        plsc.subcore_barrier()
        for kk in range(3):
            off = sid * SLC + kk * ZB
            pltpu.sync_copy(acc.at[pl.ds(off, ZB)],
                            out_h.at[pl.ds(cid * ACC + off, ZB)])

    return k(es, ed, xf, consts, sd2)


def _tc_finalize(xpad, accs, W1, as1, ad1, b1, W2, as2, ad2, b2, Wl, bl, sd):
    def body(x_ref, acc_ref, w1_ref, as1_ref, ad1_ref, b1_ref, w2_ref,
             as2_ref, ad2_ref, b2_ref, wl_ref, bl_ref, sd_ref, out_ref):
        x = x_ref[...]
        X = jnp.max(jnp.abs(x))
        cs = []
        cd = []
        for h in range(2):
            cs.append(sum(w1_ref[0, 4 * h + c] * as1_ref[h, c] for c in range(4)))
            cd.append(sum(w1_ref[0, 4 * h + c] * ad1_ref[h, c] for c in range(4)))

        acc = acc_ref[0] + acc_ref[1]
        rr = []
        for h in range(2):
            arg_s = x * (cs[h] + cd[h])
            e_self = jnp.maximum(arg_s, 0.2 * arg_s)
            ma = X * jnp.abs(cs[h]) + x * cd[h]
            m = jnp.maximum(ma, 0.2 * ma)
            p = jnp.exp(e_self - m)
            den = acc[h] + p
            t = acc[2 + h] + p * x
            rr.append(t / (den + 1e-16))
        h1 = [w1_ref[0, j] * rr[j // 4] + b1_ref[0, j] for j in range(8)]
        z2 = [sum(h1[kk] * w2_ref[kk, j] for kk in range(8)) for j in range(8)]
        als = [sum(z2[4 * h + c] * as2_ref[h, c] for c in range(4))
               for h in range(2)]

        r_iota = lax.broadcasted_iota(jnp.int32, (RP, 128), 0)
        c_iota = lax.broadcasted_iota(jnp.int32, (RP, 128), 1)
        vid = r_iota * 128 + c_iota
        valid = vid < N

        h2 = []
        for ti in range(2):
            t_id = sd_ref[0, ti]
            cnt = acc[4 + ti]
            is_t = (vid == t_id).astype(jnp.float32)
            cnt_eff = cnt + is_t
            msk = (cnt_eff > 0.0) & valid
            z2t = [jnp.sum(jnp.where(vid == t_id, z2[j], 0.0)) for j in range(8)]
            for h in range(2):
                aldt = sum(z2t[4 * h + c] * ad2_ref[h, c] for c in range(4))
                arg = als[h] + aldt
                e2 = jnp.maximum(arg, 0.2 * arg)
                M2 = jnp.max(jnp.where(msk, e2, -1e30))
                w = jnp.where(msk, cnt_eff * jnp.exp(e2 - M2), 0.0)
                den2 = jnp.sum(w)
                for c in range(4):
                    num = jnp.sum(w * z2[4 * h + c])
                    h2.append(num / (den2 + 1e-16) + b2_ref[0, 4 * h + c])
        y = [sum(h2[j] * wl_ref[j, kk] for j in range(16)) + bl_ref[0, kk]
             for kk in range(2)]
        out_ref[...] = jnp.stack(y).reshape(1, 2)

    return pl.pallas_call(
        body,
        out_shape=jax.ShapeDtypeStruct((1, 2), jnp.float32),
    )(xpad, accs, W1, as1, ad1, b1, W2, as2, ad2, b2, Wl, bl, sd)


def kernel(x, edge_index, src, dst, W1, a_src1, a_dst1, b1,
           W2, a_src2, a_dst2, b2, Wl, bl):
    xf = x.reshape(N).astype(jnp.float32)
    xpad = jnp.concatenate([xf, jnp.zeros((NP - N,), jnp.float32)])
    xpad2 = xpad.reshape(RP, 128)
    es = edge_index[0].astype(jnp.int32)
    ed = edge_index[1].astype(jnp.int32)
    srci = jnp.asarray(src, jnp.int32)
    dsti = jnp.asarray(dst, jnp.int32)
    sd2 = jnp.stack([jnp.full((16,), srci), jnp.full((16,), dsti)])
    sd = jnp.stack([srci, dsti]).reshape(1, 2)

    consts = _tc_prep(xpad2, W1, a_src1, a_dst1)
    acc_flat = _sc_edges(es, ed, xpad, consts, sd2)
    accs = acc_flat.reshape(2, 6, RP, 128)
    y = _tc_finalize(xpad2, accs, W1, a_src1, a_dst1, b1.reshape(1, 8),
                     W2, a_src2, a_dst2, b2.reshape(1, 8),
                     Wl, bl.reshape(1, 2), sd)
    return y.reshape(2)
